# baseline (speedup 1.0000x reference)
import functools
import math

import jax
import jax.numpy as jnp
from jax import lax
from jax.experimental import pallas as pl
from jax.experimental.pallas import tpu as pltpu
from jax.experimental.pallas import tpu_sc as plsc

D_MODEL = 64
SEQ_LEN = 2048
BATCH = 4
NC = 2
NS = 16
NW = NC * NS
ROWS_PER_W = (SEQ_LEN * BATCH) // NW
CHUNK = 128
SEQ_PER_W = ROWS_PER_W // BATCH
SEQ_PER_CHUNK = CHUNK // BATCH
LANES = 16
VPD = D_MODEL // LANES


def _fma_chunk(rows, pe_v, pe_off):
    def body(s, carry):
        pvals = [pe_v[pe_off + s, pl.ds(j * LANES, LANES)] for j in range(VPD)]
        for b in range(BATCH):
            r = s * BATCH + b
            for j in range(VPD):
                sl = pl.ds(j * LANES, LANES)
                rows[r, sl] = rows[r, sl] * 8.0 + pvals[j]
        return carry

    lax.fori_loop(0, SEQ_PER_CHUNK, body, 0, unroll=False)


def _emb_body(w_hbm, x_hbm, pe_hbm, out_hbm, idx_v, rows0, rows1, pe_v,
              sem0, sem1, sem2):
    wid = lax.axis_index("s") * NC + lax.axis_index("c")
    base = wid * ROWS_PER_W

    pltpu.sync_copy(x_hbm.at[pl.ds(2 * wid, 2)], idx_v)
    g0 = pltpu.async_copy(w_hbm.at[idx_v.at[0]], rows0, sem0)
    g1 = pltpu.async_copy(w_hbm.at[idx_v.at[1]], rows1, sem1)
    pltpu.sync_copy(pe_hbm.at[pl.ds(wid * SEQ_PER_W, SEQ_PER_W)], pe_v)

    g0.wait()
    _fma_chunk(rows0, pe_v, 0)
    st0 = pltpu.async_copy(rows0, out_hbm.at[pl.ds(base, CHUNK)], sem2)
    g1.wait()
    _fma_chunk(rows1, pe_v, SEQ_PER_CHUNK)
    pltpu.sync_copy(rows1, out_hbm.at[pl.ds(base + CHUNK, CHUNK)])
    st0.wait()


_emb_lookup = functools.partial(
    pl.kernel,
    out_type=jax.ShapeDtypeStruct((SEQ_LEN * BATCH, D_MODEL), jnp.float32),
    mesh=plsc.VectorSubcoreMesh(core_axis_name="c", subcore_axis_name="s"),
    scratch_types=[
        pltpu.VMEM((2, CHUNK), jnp.int32),
        pltpu.VMEM((CHUNK, D_MODEL), jnp.float32),
        pltpu.VMEM((CHUNK, D_MODEL), jnp.float32),
        pltpu.VMEM((SEQ_PER_W, D_MODEL), jnp.float32),
        pltpu.SemaphoreType.DMA,
        pltpu.SemaphoreType.DMA,
        pltpu.SemaphoreType.DMA,
    ],
    compiler_params=pltpu.CompilerParams(use_tc_tiling_on_sc=False),
)(_emb_body)


@jax.jit
def kernel(x, weight, pe):
    s, b = x.shape
    d = weight.shape[1]
    x2d = x.reshape(-1).astype(jnp.int32).reshape(NW * 2, CHUNK)
    pe2d = pe[:s, 0, :]
    out = _emb_lookup(weight, x2d, pe2d)
    return out.reshape(s, b, d)
